# baseline (speedup 1.0000x reference)
import jax
import jax.numpy as jnp
from jax.experimental import pallas as pl
from jax.experimental.pallas import tpu as pltpu

K = 64
RB = 128
N_ITERS = 22
TW = 512
_BIG = 3.0e38


def _body(x_ref, wt_ref, z_ref, s_ref, mn_ref, mx_ref):
    i = pl.program_id(0)
    cur = jax.lax.rem(i, 2)
    prv = 1 - cur
    nt = wt_ref.shape[1] // TW

    x = x_ref[...]
    xn = x * jax.lax.rsqrt(jnp.maximum(jnp.sum(x * x, axis=1, keepdims=True),
                                       1e-24))

    lo0 = mn_ref[prv, :, :1] - 1e-6
    hi0 = mx_ref[prv, :, :1] + 1e-6

    def step(it, carry):
        lo, hi, bmn, bmx = carry
        t = jnp.minimum(it, nt - 1)
        sl = pl.ds(t * TW, TW)
        tile = jnp.dot(xn, wt_ref[:, sl], preferred_element_type=jnp.float32)
        s_ref[cur, :, sl] = tile
        bmn = jnp.minimum(bmn, jnp.min(tile, axis=1, keepdims=True))
        bmx = jnp.maximum(bmx, jnp.max(tile, axis=1, keepdims=True))
        mid = (lo + hi) * 0.5
        cnt = jnp.sum((s_ref[prv] >= mid).astype(jnp.float32), axis=1,
                      keepdims=True)
        ge = cnt >= K
        return jnp.where(ge, mid, lo), jnp.where(ge, hi, mid), bmn, bmx

    lo, hi, bmn, bmx = jax.lax.fori_loop(
        0, N_ITERS, step,
        (lo0, hi0, jnp.full((RB, 1), _BIG, jnp.float32),
         jnp.full((RB, 1), -_BIG, jnp.float32)))

    mn_ref[cur] = jnp.broadcast_to(bmn, (RB, 128))
    mx_ref[cur] = jnp.broadcast_to(bmx, (RB, 128))
    sp = s_ref[prv]
    z_ref[...] = jnp.where(sp >= lo, jnp.maximum(sp, 0.0), 0.0)


def kernel(x, W):
    B, D = x.shape
    N = W.shape[0]
    nb = B // RB
    wt = W.T
    return pl.pallas_call(
        _body,
        grid=(nb + 1,),
        in_specs=[
            pl.BlockSpec((RB, D), lambda i: (jnp.minimum(i, nb - 1), 0)),
            pl.BlockSpec((D, N), lambda i: (0, 0)),
        ],
        out_specs=pl.BlockSpec((RB, N), lambda i: (jnp.maximum(i - 1, 0), 0)),
        out_shape=jax.ShapeDtypeStruct((B, N), jnp.float32),
        scratch_shapes=[
            pltpu.VMEM((2, RB, N), jnp.float32),
            pltpu.VMEM((2, RB, 128), jnp.float32),
            pltpu.VMEM((2, RB, 128), jnp.float32),
        ],
    )(x, wt)


# speedup vs baseline: 1.7765x; 1.7765x over previous
import jax
import jax.numpy as jnp
from jax.experimental import pallas as pl
from jax.experimental.pallas import tpu as pltpu

K = 64
RB = 128
N_ITERS = 18


def _body(x_ref, wt_ref, z_ref, s_ref):
    x = x_ref[...]
    xn = x * jax.lax.rsqrt(jnp.maximum(jnp.sum(x * x, axis=1, keepdims=True),
                                       1e-24))
    s = jnp.dot(xn, wt_ref[...], preferred_element_type=jnp.float32)
    s_ref[...] = s

    hi = jnp.max(s, axis=1, keepdims=True) + 1e-6
    lo = jnp.zeros_like(hi)

    def step(_, carry):
        lo, hi = carry
        mid = (lo + hi) * 0.5
        cnt = jnp.sum((s_ref[...] >= mid).astype(jnp.float32), axis=1,
                      keepdims=True)
        ge = cnt >= K
        return jnp.where(ge, mid, lo), jnp.where(ge, hi, mid)

    lo, hi = jax.lax.fori_loop(0, N_ITERS, step, (lo, hi))
    s = s_ref[...]
    z_ref[...] = jnp.where(s >= lo, jnp.maximum(s, 0.0), 0.0)


def kernel(x, W):
    B, D = x.shape
    N = W.shape[0]
    wt = W.T
    return pl.pallas_call(
        _body,
        grid=(B // RB,),
        in_specs=[
            pl.BlockSpec((RB, D), lambda i: (i, 0)),
            pl.BlockSpec((D, N), lambda i: (0, 0)),
        ],
        out_specs=pl.BlockSpec((RB, N), lambda i: (i, 0)),
        out_shape=jax.ShapeDtypeStruct((B, N), jnp.float32),
        scratch_shapes=[pltpu.VMEM((RB, N), jnp.float32)],
    )(x, wt)
